# baseline (speedup 1.0000x reference)
import functools

import jax
import jax.numpy as jnp
from jax import lax
from jax.experimental import pallas as pl
from jax.experimental.pallas import tpu as pltpu
from jax.experimental.pallas import tpu_sc as plsc

_NUM_CORES = 2
_NUM_SUBCORES = 16
_NW = _NUM_CORES * _NUM_SUBCORES
_LANES = 16
_NBUF = 4
_BLK = 32768


def _project_table(tbl_t, w_pad, V):
    grid = (V + _BLK - 1) // _BLK

    def body(w_ref, t_ref, p0_ref, p1_ref):
        m = jnp.dot(w_ref[...], t_ref[...], preferred_element_type=jnp.float32)
        p0_ref[...] = m[0]
        p1_ref[...] = m[1]

    return pl.pallas_call(
        body,
        grid=(grid,),
        in_specs=[
            pl.BlockSpec((8, 64), lambda g: (0, 0)),
            pl.BlockSpec((64, _BLK), lambda g: (0, g)),
        ],
        out_specs=[
            pl.BlockSpec((_BLK,), lambda g: (g,)),
            pl.BlockSpec((_BLK,), lambda g: (g,)),
        ],
        out_shape=[
            jax.ShapeDtypeStruct((V,), jnp.float32),
            jax.ShapeDtypeStruct((V,), jnp.float32),
        ],
    )(w_pad, tbl_t)


def _sc_pool(seq, lengths, p01, fc_b_pad, L, B):
    BPW = B // _NW

    mesh = plsc.VectorSubcoreMesh(core_axis_name="c", subcore_axis_name="s")

    @functools.partial(
        pl.kernel,
        mesh=mesh,
        out_type=jax.ShapeDtypeStruct((B * 2,), jnp.float32),
        compiler_params=pltpu.CompilerParams(
            needs_layout_passes=False, use_tc_tiling_on_sc=False),
        scratch_types=(
            [pltpu.VMEM((L, BPW), jnp.int32)]
            + [pltpu.VMEM((BPW * L,), jnp.int32)]
            + [pltpu.VMEM((L, 2), jnp.float32) for _ in range(_NBUF)]
            + [pltpu.VMEM((BPW,), jnp.int32)]
            + [pltpu.VMEM((BPW,), jnp.float32)]
            + [pltpu.VMEM((_LANES,), jnp.float32)]
            + [pltpu.VMEM((BPW * 2,), jnp.float32)]
            + [pltpu.SemaphoreType.DMA for _ in range(_NBUF)]
        ),
    )
    def body(seq_hbm, len_hbm, p01_hbm, b_hbm, out_hbm,
             tmp_v, idx_v, d0, d1, d2, d3,
             len_v, invlen_v, b_v, out_v,
             s0, s1, s2, s3):
        dbuf = (d0, d1, d2, d3)
        sems = (s0, s1, s2, s3)
        wid = lax.axis_index("s") * _NUM_CORES + lax.axis_index("c")
        base = wid * BPW

        pltpu.sync_copy(seq_hbm.at[:, pl.ds(base, BPW)], tmp_v)
        pltpu.sync_copy(len_hbm.at[pl.ds(base, BPW)], len_v)
        pltpu.sync_copy(b_hbm, b_v)

        lane = jnp.arange(_LANES, dtype=jnp.int32)
        scatter_bases = [(lane + c * _LANES) * L for c in range(BPW // _LANES)]

        def transpose_t(t, carry):
            tvec = jnp.full((_LANES,), t, jnp.int32)
            for c in range(BPW // _LANES):
                v = tmp_v[t, pl.ds(c * _LANES, _LANES)]
                plsc.store_scatter(idx_v, [scatter_bases[c] + tvec], v)
            return carry

        lax.fori_loop(0, L, transpose_t, 0)

        def start_gather(g, slot):
            idx = idx_v.at[pl.ds(g * L, L)]
            pltpu.make_async_copy(p01_hbm.at[idx], dbuf[slot], sems[slot]).start()

        def wait_gather(g, slot):
            idx = idx_v.at[pl.ds(g * L, L)]
            pltpu.make_async_copy(p01_hbm.at[idx], dbuf[slot], sems[slot]).wait()

        bias_vec = b_v[:]
        zero = jnp.zeros((_LANES,), jnp.float32)
        n_chunks2 = (L * 2) // _LANES
        row_base = jnp.arange(_LANES, dtype=jnp.int32) // 2
        col_idx = jnp.arange(_LANES, dtype=jnp.int32) % 2
        even_mask = col_idx == 0

        for k in range(BPW // _LANES):
            lv = len_v[pl.ds(k * _LANES, _LANES)].astype(jnp.float32)
            invlen_v[pl.ds(k * _LANES, _LANES)] = 1.0 / lv

        for s in range(_NBUF):
            start_gather(s, s)

        def reduce_both(buf):
            acc = plsc.load_gather(buf, [row_base, col_idx])
            for c in range(1, n_chunks2):
                acc = acc + plsc.load_gather(buf, [row_base + c * 8, col_idx])
            sum0 = plsc.cumsum(jnp.where(even_mask, acc, 0.0))[_LANES - 1]
            sum1 = plsc.cumsum(jnp.where(even_mask, 0.0, acc))[_LANES - 1]
            return sum0, sum1

        def outer(j, carry):
            iv = invlen_v[pl.ds(j * _LANES, _LANES)]
            packed = zero
            for b in range(_LANES):
                g = j * _LANES + b
                slot = b % _NBUF
                wait_gather(g, slot)
                sum0, sum1 = reduce_both(dbuf[slot])
                nxt = g + _NBUF

                @pl.when(nxt < BPW)
                def _():
                    start_gather(nxt, slot)

                inv = jnp.full((_LANES,), iv[b])
                sel = jnp.where(lane == 0, sum0, jnp.where(lane == 1, sum1, 0.0))
                sig = 1.0 / (1.0 + jnp.exp(-(sel * inv + bias_vec)))
                k = b % 8
                packed = jnp.where(lane == 2 * k, sig[0],
                                   jnp.where(lane == 2 * k + 1, sig[1], packed))
                if k == 7:
                    out_v[pl.ds(j * 2 * _LANES + (b // 8) * _LANES, _LANES)] = packed
                    packed = zero
            return carry

        lax.fori_loop(0, BPW // _LANES, outer, 0)
        pltpu.sync_copy(out_v, out_hbm.at[pl.ds(base * 2, BPW * 2)])

    return body(seq, lengths, p01, fc_b_pad)


def kernel(seq, lengths, emb_table, fc_w, fc_b):
    L, B = seq.shape
    V, D = emb_table.shape
    w_pad = jnp.zeros((8, D), jnp.float32).at[:2].set(fc_w)
    fc_b_pad = jnp.zeros((_LANES,), jnp.float32).at[:2].set(fc_b)
    p0, p1 = _project_table(emb_table.T, w_pad, V)
    p01 = jnp.stack([p0, p1], axis=-1)
    out_flat = _sc_pool(seq, lengths, p01, fc_b_pad, L, B)
    return out_flat.reshape(B, 2)


# speedup vs baseline: 10.2971x; 10.2971x over previous
import functools

import jax
import jax.numpy as jnp
from jax import lax
from jax.experimental import pallas as pl
from jax.experimental.pallas import tpu as pltpu
from jax.experimental.pallas import tpu_sc as plsc

_NUM_CORES = 2
_NUM_SUBCORES = 16
_NW = _NUM_CORES * _NUM_SUBCORES
_LANES = 16
_NBUF = 4
_BLK = 32768


def _project_table(tbl_t, w_pad, V):
    grid = (V + _BLK - 1) // _BLK

    def body(w_ref, t_ref, p0_ref, p1_ref):
        m = jnp.dot(w_ref[...], t_ref[...], preferred_element_type=jnp.float32)
        p0_ref[...] = m[0]
        p1_ref[...] = m[1]

    return pl.pallas_call(
        body,
        grid=(grid,),
        in_specs=[
            pl.BlockSpec((8, 64), lambda g: (0, 0)),
            pl.BlockSpec((64, _BLK), lambda g: (0, g)),
        ],
        out_specs=[
            pl.BlockSpec((_BLK,), lambda g: (g,)),
            pl.BlockSpec((_BLK,), lambda g: (g,)),
        ],
        out_shape=[
            jax.ShapeDtypeStruct((V,), jnp.float32),
            jax.ShapeDtypeStruct((V,), jnp.float32),
        ],
    )(w_pad, tbl_t)


def _sc_pool(seq, lengths, p0, p1, fc_b_pad, L, B):
    BPW = B // _NW

    mesh = plsc.VectorSubcoreMesh(core_axis_name="c", subcore_axis_name="s")

    @functools.partial(
        pl.kernel,
        mesh=mesh,
        out_type=jax.ShapeDtypeStruct((B * 2,), jnp.float32),
        compiler_params=pltpu.CompilerParams(
            needs_layout_passes=False, use_tc_tiling_on_sc=False),
        scratch_types=(
            [pltpu.VMEM((L, BPW), jnp.int32)]
            + [pltpu.VMEM((BPW * L,), jnp.int32)]
            + [pltpu.VMEM((L,), jnp.float32) for _ in range(2 * _NBUF)]
            + [pltpu.VMEM((BPW,), jnp.int32)]
            + [pltpu.VMEM((BPW,), jnp.float32)]
            + [pltpu.VMEM((_LANES,), jnp.float32)]
            + [pltpu.VMEM((BPW * 2,), jnp.float32)]
            + [pltpu.SemaphoreType.DMA for _ in range(2 * _NBUF)]
        ),
    )
    def body(seq_hbm, len_hbm, p0_hbm, p1_hbm, b_hbm, out_hbm,
             tmp_v, idx_v, d00, d01, d02, d03, d10, d11, d12, d13,
             len_v, invlen_v, b_v, out_v,
             s00, s01, s02, s03, s10, s11, s12, s13):
        d0 = (d00, d01, d02, d03)
        d1 = (d10, d11, d12, d13)
        sm0 = (s00, s01, s02, s03)
        sm1 = (s10, s11, s12, s13)
        wid = lax.axis_index("s") * _NUM_CORES + lax.axis_index("c")
        base = wid * BPW

        pltpu.sync_copy(seq_hbm.at[:, pl.ds(base, BPW)], tmp_v)
        pltpu.sync_copy(len_hbm.at[pl.ds(base, BPW)], len_v)
        pltpu.sync_copy(b_hbm, b_v)

        lane = jnp.arange(_LANES, dtype=jnp.int32)
        scatter_bases = [(lane + c * _LANES) * L for c in range(BPW // _LANES)]

        def transpose_t(t, carry):
            tvec = jnp.full((_LANES,), t, jnp.int32)
            for c in range(BPW // _LANES):
                v = tmp_v[t, pl.ds(c * _LANES, _LANES)]
                plsc.store_scatter(idx_v, [scatter_bases[c] + tvec], v)
            return carry

        lax.fori_loop(0, L, transpose_t, 0)

        def start_gather(g, slot):
            idx = idx_v.at[pl.ds(g * L, L)]
            pltpu.make_async_copy(p0_hbm.at[idx], d0[slot], sm0[slot]).start()
            pltpu.make_async_copy(p1_hbm.at[idx], d1[slot], sm1[slot]).start()

        def wait_gather(g, slot):
            idx = idx_v.at[pl.ds(g * L, L)]
            pltpu.make_async_copy(p0_hbm.at[idx], d0[slot], sm0[slot]).wait()
            pltpu.make_async_copy(p1_hbm.at[idx], d1[slot], sm1[slot]).wait()

        bias_vec = b_v[:]
        zero = jnp.zeros((_LANES,), jnp.float32)
        n_full = L // _LANES
        tail = L - n_full * _LANES
        tail_start = L - _LANES

        for k in range(BPW // _LANES):
            lv = len_v[pl.ds(k * _LANES, _LANES)].astype(jnp.float32)
            invlen_v[pl.ds(k * _LANES, _LANES)] = 1.0 / lv

        for s in range(_NBUF):
            start_gather(s, s)

        def reduce_ch(buf):
            acc = buf[pl.ds(0, _LANES)]
            for c in range(1, n_full):
                acc = acc + buf[pl.ds(c * _LANES, _LANES)]
            if tail:
                tv = buf[pl.ds(tail_start, _LANES)]
                acc = acc + jnp.where(lane >= _LANES - tail, tv, 0.0)
            return plsc.cumsum(acc)[_LANES - 1]

        def outer(j, carry):
            iv = invlen_v[pl.ds(j * _LANES, _LANES)]
            packed = zero
            for b in range(_LANES):
                g = j * _LANES + b
                slot = b % _NBUF
                wait_gather(g, slot)
                sum0 = reduce_ch(d0[slot])
                sum1 = reduce_ch(d1[slot])
                nxt = g + _NBUF

                @pl.when(nxt < BPW)
                def _():
                    start_gather(nxt, slot)

                inv = jnp.full((_LANES,), iv[b])
                sel = jnp.where(lane == 0, sum0, jnp.where(lane == 1, sum1, 0.0))
                sig = 1.0 / (1.0 + jnp.exp(-(sel * inv + bias_vec)))
                k = b % 8
                packed = jnp.where(lane == 2 * k, sig[0],
                                   jnp.where(lane == 2 * k + 1, sig[1], packed))
                if k == 7:
                    out_v[pl.ds(j * 2 * _LANES + (b // 8) * _LANES, _LANES)] = packed
                    packed = zero
            return carry

        lax.fori_loop(0, BPW // _LANES, outer, 0)
        pltpu.sync_copy(out_v, out_hbm.at[pl.ds(base * 2, BPW * 2)])

    return body(seq, lengths, p0, p1, fc_b_pad)


def kernel(seq, lengths, emb_table, fc_w, fc_b):
    L, B = seq.shape
    V, D = emb_table.shape
    w_pad = jnp.zeros((8, D), jnp.float32).at[:2].set(fc_w)
    fc_b_pad = jnp.zeros((_LANES,), jnp.float32).at[:2].set(fc_b)
    p0, p1 = _project_table(emb_table.T, w_pad, V)
    out_flat = _sc_pool(seq, lengths, p0, p1, fc_b_pad, L, B)
    return out_flat.reshape(B, 2)


# speedup vs baseline: 10.6184x; 1.0312x over previous
import functools

import jax
import jax.numpy as jnp
from jax import lax
from jax.experimental import pallas as pl
from jax.experimental.pallas import tpu as pltpu
from jax.experimental.pallas import tpu_sc as plsc

_NUM_CORES = 2
_NUM_SUBCORES = 16
_NW = _NUM_CORES * _NUM_SUBCORES
_LANES = 16
_NBUF = 4
_BLK = 32768


def _project_table(tbl_t, w_pad, V):
    grid = (V + _BLK - 1) // _BLK

    def body(w_ref, t_ref, p0_ref, p1_ref):
        m = jnp.dot(w_ref[...], t_ref[...], preferred_element_type=jnp.float32)
        p0_ref[...] = m[0]
        p1_ref[...] = m[1]

    return pl.pallas_call(
        body,
        grid=(grid,),
        in_specs=[
            pl.BlockSpec((8, 64), lambda g: (0, 0)),
            pl.BlockSpec((64, _BLK), lambda g: (0, g)),
        ],
        out_specs=[
            pl.BlockSpec((_BLK,), lambda g: (g,)),
            pl.BlockSpec((_BLK,), lambda g: (g,)),
        ],
        out_shape=[
            jax.ShapeDtypeStruct((V,), jnp.float32),
            jax.ShapeDtypeStruct((V,), jnp.float32),
        ],
    )(w_pad, tbl_t)


def _sc_prep(seq, lengths, L, B):
    BPW = B // _NW

    mesh = plsc.VectorSubcoreMesh(core_axis_name="c", subcore_axis_name="s")

    @functools.partial(
        pl.kernel,
        mesh=mesh,
        out_type=[
            jax.ShapeDtypeStruct((B * L,), jnp.int32),
            jax.ShapeDtypeStruct((B,), jnp.float32),
        ],
        compiler_params=pltpu.CompilerParams(
            needs_layout_passes=False, use_tc_tiling_on_sc=False),
        scratch_types=(
            [pltpu.VMEM((L, BPW), jnp.int32)]
            + [pltpu.VMEM((BPW * L,), jnp.int32)]
            + [pltpu.VMEM((BPW,), jnp.int32)]
            + [pltpu.VMEM((BPW,), jnp.float32)]
        ),
    )
    def prep(seq_hbm, len_hbm, idx_out, inv_out, tmp_v, idx_v, len_v, invlen_v):
        wid = lax.axis_index("s") * _NUM_CORES + lax.axis_index("c")
        base = wid * BPW

        pltpu.sync_copy(seq_hbm.at[:, pl.ds(base, BPW)], tmp_v)
        pltpu.sync_copy(len_hbm.at[pl.ds(base, BPW)], len_v)

        lane = jnp.arange(_LANES, dtype=jnp.int32)
        scatter_bases = [(lane + c * _LANES) * L for c in range(BPW // _LANES)]

        def transpose_t(t, carry):
            tvec = jnp.full((_LANES,), t, jnp.int32)
            for c in range(BPW // _LANES):
                v = tmp_v[t, pl.ds(c * _LANES, _LANES)]
                plsc.store_scatter(idx_v, [scatter_bases[c] + tvec], v)
            return carry

        lax.fori_loop(0, L, transpose_t, 0)

        for k in range(BPW // _LANES):
            lv = len_v[pl.ds(k * _LANES, _LANES)].astype(jnp.float32)
            invlen_v[pl.ds(k * _LANES, _LANES)] = 1.0 / lv

        pltpu.sync_copy(idx_v, idx_out.at[pl.ds(base * L, BPW * L)])
        pltpu.sync_copy(invlen_v, inv_out.at[pl.ds(base, BPW)])

    return prep(seq, lengths)


def _sc_pool(idx_all, inv_all, p0, p1, fc_b_pad, L, B):
    BPW = B // _NW

    mesh = plsc.VectorSubcoreMesh(core_axis_name="c", subcore_axis_name="s")

    @functools.partial(
        pl.kernel,
        mesh=mesh,
        out_type=jax.ShapeDtypeStruct((B * 2,), jnp.float32),
        compiler_params=pltpu.CompilerParams(
            needs_layout_passes=False, use_tc_tiling_on_sc=False),
        scratch_types=(
            [pltpu.VMEM((BPW * L,), jnp.int32)]
            + [pltpu.VMEM((L,), jnp.float32) for _ in range(2 * _NBUF)]
            + [pltpu.VMEM((BPW,), jnp.float32)]
            + [pltpu.VMEM((_LANES,), jnp.float32)]
            + [pltpu.VMEM((BPW * 2,), jnp.float32)]
            + [pltpu.SemaphoreType.DMA for _ in range(2 * _NBUF)]
        ),
    )
    def body(idx_hbm, inv_hbm, p0_hbm, p1_hbm, b_hbm, out_hbm,
             idx_v, d00, d01, d02, d03, d10, d11, d12, d13,
             invlen_v, b_v, out_v,
             s00, s01, s02, s03, s10, s11, s12, s13):
        d0 = (d00, d01, d02, d03)
        d1 = (d10, d11, d12, d13)
        sm0 = (s00, s01, s02, s03)
        sm1 = (s10, s11, s12, s13)
        wid = lax.axis_index("s") * _NUM_CORES + lax.axis_index("c")
        base = wid * BPW

        pltpu.sync_copy(idx_hbm.at[pl.ds(base * L, BPW * L)], idx_v)
        pltpu.sync_copy(inv_hbm.at[pl.ds(base, BPW)], invlen_v)
        pltpu.sync_copy(b_hbm, b_v)
        lane = jnp.arange(_LANES, dtype=jnp.int32)

        def start_gather(g, slot):
            idx = idx_v.at[pl.ds(g * L, L)]
            pltpu.make_async_copy(p0_hbm.at[idx], d0[slot], sm0[slot]).start()
            pltpu.make_async_copy(p1_hbm.at[idx], d1[slot], sm1[slot]).start()

        def wait_gather(g, slot):
            idx = idx_v.at[pl.ds(g * L, L)]
            pltpu.make_async_copy(p0_hbm.at[idx], d0[slot], sm0[slot]).wait()
            pltpu.make_async_copy(p1_hbm.at[idx], d1[slot], sm1[slot]).wait()

        bias_vec = b_v[:]
        zero = jnp.zeros((_LANES,), jnp.float32)
        n_full = L // _LANES
        tail = L - n_full * _LANES
        tail_start = L - _LANES

        for s in range(_NBUF):
            start_gather(s, s)

        def reduce_ch(buf):
            acc = buf[pl.ds(0, _LANES)]
            for c in range(1, n_full):
                acc = acc + buf[pl.ds(c * _LANES, _LANES)]
            if tail:
                tv = buf[pl.ds(tail_start, _LANES)]
                acc = acc + jnp.where(lane >= _LANES - tail, tv, 0.0)
            return plsc.cumsum(acc)[_LANES - 1]

        def outer(j, carry):
            iv = invlen_v[pl.ds(j * _LANES, _LANES)]
            packed = zero
            for b in range(_LANES):
                g = j * _LANES + b
                slot = b % _NBUF
                wait_gather(g, slot)
                sum0 = reduce_ch(d0[slot])
                sum1 = reduce_ch(d1[slot])
                nxt = g + _NBUF

                @pl.when(nxt < BPW)
                def _():
                    start_gather(nxt, slot)

                inv = jnp.full((_LANES,), iv[b])
                sel = jnp.where(lane == 0, sum0, jnp.where(lane == 1, sum1, 0.0))
                sig = 1.0 / (1.0 + jnp.exp(-(sel * inv + bias_vec)))
                k = b % 8
                packed = jnp.where(lane == 2 * k, sig[0],
                                   jnp.where(lane == 2 * k + 1, sig[1], packed))
                if k == 7:
                    out_v[pl.ds(j * 2 * _LANES + (b // 8) * _LANES, _LANES)] = packed
                    packed = zero
            return carry

        lax.fori_loop(0, BPW // _LANES, outer, 0)
        pltpu.sync_copy(out_v, out_hbm.at[pl.ds(base * 2, BPW * 2)])

    return body(idx_all, inv_all, p0, p1, fc_b_pad)


def kernel(seq, lengths, emb_table, fc_w, fc_b):
    L, B = seq.shape
    V, D = emb_table.shape
    w_pad = jnp.zeros((8, D), jnp.float32).at[:2].set(fc_w)
    fc_b_pad = jnp.zeros((_LANES,), jnp.float32).at[:2].set(fc_b)
    idx_all, inv_all = _sc_prep(seq, lengths, L, B)
    p0, p1 = _project_table(emb_table.T, w_pad, V)
    out_flat = _sc_pool(idx_all, inv_all, p0, p1, fc_b_pad, L, B)
    return out_flat.reshape(B, 2)
